# Initial kernel scaffold; baseline (speedup 1.0000x reference)
import jax
import jax.numpy as jnp
from jax.experimental import pallas as pl


def kernel(features_p2, features_p3, features_p4, features_p5, features_p6, conv_w, conv_b, obj_w, obj_b, delta_w, delta_b):
    raise NotImplementedError("write your pallas kernel here")


# speedup vs baseline: 72.9891x; 72.9891x over previous
import functools
import math

import numpy as np
import jax
import jax.numpy as jnp
from jax.experimental import pallas as pl
from jax.experimental.pallas import tpu as pltpu

_N = 2
_C = 256
_A = 3
_IMG = 512
_STRIDES = [4, 8, 16, 32, 64]
_SIZES = [32.0, 64.0, 128.0, 256.0, 512.0]
_RATIOS = [0.5, 1.0, 2.0]
_HWS = [(_IMG // s, _IMG // s) for s in _STRIDES]
_PRE_NMS_TOPK = 6000
_POST_NMS_TOPK = 1000
_NMS_THRESH = 0.7
_SCALE_CLAMP = math.log(1000.0 / 16)
_B = 128
_INTERPRET = False


def _cell_anchors_np(size):
    out = []
    area = size * size
    for r in _RATIOS:
        w = math.sqrt(area / r)
        h = w * r
        out.append([-w / 2.0, -h / 2.0, w / 2.0, h / 2.0])
    return np.asarray(out, dtype=np.float32)


def _grid_anchors_np():
    anchors = []
    for (H, W), stride, size in zip(_HWS, _STRIDES, _SIZES):
        ca = _cell_anchors_np(size)
        sx = np.arange(W, dtype=np.float32) * stride
        sy = np.arange(H, dtype=np.float32) * stride
        yy, xx = np.meshgrid(sy, sx, indexing='ij')
        shifts = np.stack([xx.ravel(), yy.ravel(), xx.ravel(), yy.ravel()], axis=1)
        anchors.append((shifts[:, None, :] + ca[None, :, :]).reshape(-1, 4))
    return anchors


def _conv(x, w, b, pad):
    y = jax.lax.conv_general_dilated(x, w, (1, 1), pad,
                                     dimension_numbers=('NCHW', 'OIHW', 'NCHW'))
    return y + b[None, :, None, None]


def _decode(anchors, deltas):
    widths = anchors[:, 2] - anchors[:, 0]
    heights = anchors[:, 3] - anchors[:, 1]
    ctr_x = anchors[:, 0] + 0.5 * widths
    ctr_y = anchors[:, 1] + 0.5 * heights
    dx, dy, dw, dh = deltas[:, 0], deltas[:, 1], deltas[:, 2], deltas[:, 3]
    dw = jnp.minimum(dw, _SCALE_CLAMP)
    dh = jnp.minimum(dh, _SCALE_CLAMP)
    pcx = dx * widths + ctr_x
    pcy = dy * heights + ctr_y
    pw = jnp.exp(dw) * widths
    ph = jnp.exp(dh) * heights
    return jnp.stack([pcx - 0.5 * pw, pcy - 0.5 * ph, pcx + 0.5 * pw, pcy + 0.5 * ph], axis=1)


def _nms_kernel_fn(boxes_ref, bT_ref, keep_ref, *, nb):
    M = bT_ref.shape[2]
    x1 = bT_ref[0, 0:1, :]
    y1 = bT_ref[0, 1:2, :]
    x2 = bT_ref[0, 2:3, :]
    y2 = bT_ref[0, 3:4, :]
    area = (x2 - x1) * (y2 - y1)
    colidx = jax.lax.broadcasted_iota(jnp.int32, (1, M), 1)
    r0 = jax.lax.broadcasted_iota(jnp.int32, (_B, _B), 0)
    r1 = jax.lax.broadcasted_iota(jnp.int32, (_B, _B), 1)
    lt = r1 < r0
    ut = r0 < r1
    eye = (r0 == r1).astype(jnp.float32)

    def body(i, carry):
        base = i * _B
        blk = boxes_ref[0, pl.ds(base, _B), :]
        bx1 = blk[:, 0:1]
        by1 = blk[:, 1:2]
        bx2 = blk[:, 2:3]
        by2 = blk[:, 3:4]
        barea = (bx2 - bx1) * (by2 - by1)
        xx1 = jnp.maximum(bx1, x1)
        yy1 = jnp.maximum(by1, y1)
        xx2 = jnp.minimum(bx2, x2)
        yy2 = jnp.minimum(by2, y2)
        inter = jnp.maximum(xx2 - xx1, 0.0) * jnp.maximum(yy2 - yy1, 0.0)
        iou = inter / (barea + area - inter + 1e-9)
        keep_now = keep_ref[0, 0:1, :] > 0.5
        se = (iou > _NMS_THRESH) & (colidx < base) & keep_now
        s_col = jnp.any(se, axis=1, keepdims=True)
        x1b = bT_ref[0, 0:1, pl.ds(base, _B)]
        y1b = bT_ref[0, 1:2, pl.ds(base, _B)]
        x2b = bT_ref[0, 2:3, pl.ds(base, _B)]
        y2b = bT_ref[0, 3:4, pl.ds(base, _B)]
        areab = (x2b - x1b) * (y2b - y1b)
        xx1b = jnp.maximum(bx1, x1b)
        yy1b = jnp.maximum(by1, y1b)
        xx2b = jnp.minimum(bx2, x2b)
        yy2b = jnp.minimum(by2, y2b)
        interb = jnp.maximum(xx2b - xx1b, 0.0) * jnp.maximum(yy2b - yy1b, 0.0)
        ioub = interb / (barea + areab - interb + 1e-9)
        pblk = ioub > _NMS_THRESH
        sf = jnp.where(s_col, 1.0, 0.0)
        ok_col_f = 1.0 - sf
        ok_row_f = jax.lax.dot_general(ok_col_f, eye, (((0,), (0,)), ((), ())),
                                       preferred_element_type=jnp.float32)

        def cond(st):
            return st[2] > 0.0

        def fix(st):
            k_row_f, k_col_f, _ = st
            k_row = k_row_f > 0.5
            k_col = k_col_f > 0.5
            sup_col = jnp.any(pblk & lt & k_row, axis=1, keepdims=True)
            sup_row = jnp.any(pblk & ut & k_col, axis=0, keepdims=True)
            k_col_n = jnp.where(sup_col, 0.0, ok_col_f)
            k_row_n = jnp.where(sup_row, 0.0, ok_row_f)
            changed = jnp.sum(jnp.abs(k_row_n - k_row_f))
            return k_row_n, k_col_n, changed

        k_row_f, k_col_f, _ = jax.lax.while_loop(
            cond, fix, (ok_row_f, ok_col_f, jnp.float32(1.0)))
        keep_ref[0, 0:1, pl.ds(base, _B)] = k_row_f
        return carry

    jax.lax.fori_loop(0, nb, body, 0)


def _nms_keep(boxes_nms):
    n, M, _ = boxes_nms.shape
    bT = jnp.transpose(boxes_nms, (0, 2, 1))
    fn = functools.partial(_nms_kernel_fn, nb=M // _B)
    keep = pl.pallas_call(
        fn,
        out_shape=jax.ShapeDtypeStruct((n, 1, M), jnp.float32),
        grid=(n,),
        in_specs=[
            pl.BlockSpec((1, M, 4), lambda i: (i, 0, 0)),
            pl.BlockSpec((1, 4, M), lambda i: (i, 0, 0)),
        ],
        out_specs=pl.BlockSpec((1, 1, M), lambda i: (i, 0, 0)),
        interpret=_INTERPRET,
    )(boxes_nms, bT)
    return keep[:, 0, :]


def _round_up(x, m):
    return (x + m - 1) // m * m


def kernel(features_p2, features_p3, features_p4, features_p5, features_p6,
           conv_w, conv_b, obj_w, obj_b, delta_w, delta_b):
    feats = [features_p2, features_p3, features_p4, features_p5, features_p6]
    anchors = _grid_anchors_np()
    all_boxes, all_scores = [], []
    for lvl, f in enumerate(feats):
        t = jax.nn.relu(_conv(f, conv_w, conv_b, 'SAME'))
        logits = _conv(t, obj_w, obj_b, 'VALID')
        deltas = _conv(t, delta_w, delta_b, 'VALID')
        n, _, H, W = logits.shape
        logits = jnp.transpose(logits, (0, 2, 3, 1)).reshape(n, -1)
        deltas = jnp.transpose(deltas.reshape(n, _A, 4, H, W),
                               (0, 3, 4, 1, 2)).reshape(n, -1, 4)
        anc = jnp.asarray(anchors[lvl])
        k = min(_PRE_NMS_TOPK, int(anc.shape[0]))
        topv, topi = jax.lax.top_k(logits, k)
        dsel = jnp.take_along_axis(deltas, topi[:, :, None], axis=1)
        asel = anc[topi]
        boxes = jax.vmap(_decode)(asel, dsel)
        boxes = jnp.clip(boxes, 0.0, float(_IMG))
        kp = _round_up(k, _B)
        if kp != k:
            boxes = jnp.pad(boxes, ((0, 0), (0, kp - k), (0, 0)))
            topv = jnp.pad(topv, ((0, 0), (0, kp - k)),
                           constant_values=-jnp.inf)
        offs = jnp.full((kp,), float(lvl), jnp.float32) * (_IMG + 1.0)
        boxes_nms = boxes + offs[None, :, None]
        keep = _nms_keep(boxes_nms)
        masked = jnp.where(keep > 0.5, topv, -jnp.inf)
        all_boxes.append(boxes)
        all_scores.append(masked)
    boxes = jnp.concatenate(all_boxes, axis=1)
    scores = jnp.concatenate(all_scores, axis=1)
    out_s, topi = jax.lax.top_k(scores, _POST_NMS_TOPK)
    out_b = jnp.take_along_axis(boxes, topi[:, :, None], axis=1)
    return out_b, out_s


# speedup vs baseline: 94.1587x; 1.2900x over previous
import functools
import math

import numpy as np
import jax
import jax.numpy as jnp
from jax.experimental import pallas as pl
from jax.experimental.pallas import tpu as pltpu

_N = 2
_C = 256
_A = 3
_IMG = 512
_STRIDES = [4, 8, 16, 32, 64]
_SIZES = [32.0, 64.0, 128.0, 256.0, 512.0]
_RATIOS = [0.5, 1.0, 2.0]
_HWS = [(_IMG // s, _IMG // s) for s in _STRIDES]
_PRE_NMS_TOPK = 6000
_POST_NMS_TOPK = 1000
_NMS_THRESH = 0.7
_SCALE_CLAMP = math.log(1000.0 / 16)
_B = 128
_INTERPRET = False


def _cell_anchors_np(size):
    out = []
    area = size * size
    for r in _RATIOS:
        w = math.sqrt(area / r)
        h = w * r
        out.append([-w / 2.0, -h / 2.0, w / 2.0, h / 2.0])
    return np.asarray(out, dtype=np.float32)


def _grid_anchors_np():
    anchors = []
    for (H, W), stride, size in zip(_HWS, _STRIDES, _SIZES):
        ca = _cell_anchors_np(size)
        sx = np.arange(W, dtype=np.float32) * stride
        sy = np.arange(H, dtype=np.float32) * stride
        yy, xx = np.meshgrid(sy, sx, indexing='ij')
        shifts = np.stack([xx.ravel(), yy.ravel(), xx.ravel(), yy.ravel()], axis=1)
        anchors.append((shifts[:, None, :] + ca[None, :, :]).reshape(-1, 4))
    return anchors


def _conv(x, w, b, pad):
    y = jax.lax.conv_general_dilated(x, w, (1, 1), pad,
                                     dimension_numbers=('NCHW', 'OIHW', 'NCHW'))
    return y + b[None, :, None, None]


def _decode(anchors, deltas):
    widths = anchors[:, 2] - anchors[:, 0]
    heights = anchors[:, 3] - anchors[:, 1]
    ctr_x = anchors[:, 0] + 0.5 * widths
    ctr_y = anchors[:, 1] + 0.5 * heights
    dx, dy, dw, dh = deltas[:, 0], deltas[:, 1], deltas[:, 2], deltas[:, 3]
    dw = jnp.minimum(dw, _SCALE_CLAMP)
    dh = jnp.minimum(dh, _SCALE_CLAMP)
    pcx = dx * widths + ctr_x
    pcy = dy * heights + ctr_y
    pw = jnp.exp(dw) * widths
    ph = jnp.exp(dh) * heights
    return jnp.stack([pcx - 0.5 * pw, pcy - 0.5 * ph, pcx + 0.5 * pw, pcy + 0.5 * ph], axis=1)


def _nms_kernel_fn(boxes_ref, bT_ref, keep_ref, *, nb):
    M = bT_ref.shape[2]
    x1 = bT_ref[0, 0:1, :]
    y1 = bT_ref[0, 1:2, :]
    x2 = bT_ref[0, 2:3, :]
    y2 = bT_ref[0, 3:4, :]
    area = (x2 - x1) * (y2 - y1)
    colidx = jax.lax.broadcasted_iota(jnp.int32, (1, M), 1)
    r0 = jax.lax.broadcasted_iota(jnp.int32, (_B, _B), 0)
    r1 = jax.lax.broadcasted_iota(jnp.int32, (_B, _B), 1)
    lt = r1 < r0
    ut = r0 < r1
    eye = (r0 == r1).astype(jnp.float32)

    def body(st):
        i, kept_cnt = st
        base = i * _B
        blk = boxes_ref[0, pl.ds(base, _B), :]
        bx1 = blk[:, 0:1]
        by1 = blk[:, 1:2]
        bx2 = blk[:, 2:3]
        by2 = blk[:, 3:4]
        barea = (bx2 - bx1) * (by2 - by1)
        xx1 = jnp.maximum(bx1, x1)
        yy1 = jnp.maximum(by1, y1)
        xx2 = jnp.minimum(bx2, x2)
        yy2 = jnp.minimum(by2, y2)
        inter = jnp.maximum(xx2 - xx1, 0.0) * jnp.maximum(yy2 - yy1, 0.0)
        iou = inter / (barea + area - inter + 1e-9)
        keep_now = keep_ref[0, 0:1, :] > 0.5
        se = (iou > _NMS_THRESH) & (colidx < base) & keep_now
        s_col = jnp.any(se, axis=1, keepdims=True)
        x1b = bT_ref[0, 0:1, pl.ds(base, _B)]
        y1b = bT_ref[0, 1:2, pl.ds(base, _B)]
        x2b = bT_ref[0, 2:3, pl.ds(base, _B)]
        y2b = bT_ref[0, 3:4, pl.ds(base, _B)]
        areab = (x2b - x1b) * (y2b - y1b)
        xx1b = jnp.maximum(bx1, x1b)
        yy1b = jnp.maximum(by1, y1b)
        xx2b = jnp.minimum(bx2, x2b)
        yy2b = jnp.minimum(by2, y2b)
        interb = jnp.maximum(xx2b - xx1b, 0.0) * jnp.maximum(yy2b - yy1b, 0.0)
        ioub = interb / (barea + areab - interb + 1e-9)
        pblk = ioub > _NMS_THRESH
        sf = jnp.where(s_col, 1.0, 0.0)
        ok_col_f = 1.0 - sf
        ok_row_f = jax.lax.dot_general(ok_col_f, eye, (((0,), (0,)), ((), ())),
                                       preferred_element_type=jnp.float32)

        def cond(st):
            return st[2] > 0.0

        def fix(st):
            k_row_f, k_col_f, _ = st
            k_row = k_row_f > 0.5
            k_col = k_col_f > 0.5
            sup_col = jnp.any(pblk & lt & k_row, axis=1, keepdims=True)
            sup_row = jnp.any(pblk & ut & k_col, axis=0, keepdims=True)
            k_col_n = jnp.where(sup_col, 0.0, ok_col_f)
            k_row_n = jnp.where(sup_row, 0.0, ok_row_f)
            changed = jnp.sum(jnp.abs(k_row_n - k_row_f))
            return k_row_n, k_col_n, changed

        k_row_f, k_col_f, _ = jax.lax.while_loop(
            cond, fix, (ok_row_f, ok_col_f, jnp.float32(1.0)))
        keep_ref[0, 0:1, pl.ds(base, _B)] = k_row_f
        return i + 1, kept_cnt + jnp.sum(k_row_f)

    jax.lax.while_loop(
        lambda st: (st[0] < nb) & (st[1] < float(_POST_NMS_TOPK)),
        body, (jnp.int32(0), jnp.float32(0.0)))


def _nms_keep(boxes_nms):
    n, M, _ = boxes_nms.shape
    bT = jnp.transpose(boxes_nms, (0, 2, 1))
    fn = functools.partial(_nms_kernel_fn, nb=M // _B)
    keep = pl.pallas_call(
        fn,
        out_shape=jax.ShapeDtypeStruct((n, 1, M), jnp.float32),
        grid=(n,),
        in_specs=[
            pl.BlockSpec((1, M, 4), lambda i: (i, 0, 0)),
            pl.BlockSpec((1, 4, M), lambda i: (i, 0, 0)),
        ],
        out_specs=pl.BlockSpec((1, 1, M), lambda i: (i, 0, 0)),
        compiler_params=pltpu.CompilerParams(
            dimension_semantics=("parallel",)),
        interpret=_INTERPRET,
    )(boxes_nms, bT)
    return keep[:, 0, :]


def _round_up(x, m):
    return (x + m - 1) // m * m


def kernel(features_p2, features_p3, features_p4, features_p5, features_p6,
           conv_w, conv_b, obj_w, obj_b, delta_w, delta_b):
    feats = [features_p2, features_p3, features_p4, features_p5, features_p6]
    anchors = _grid_anchors_np()
    all_boxes, all_scores = [], []
    for lvl, f in enumerate(feats):
        t = jax.nn.relu(_conv(f, conv_w, conv_b, 'SAME'))
        logits = _conv(t, obj_w, obj_b, 'VALID')
        deltas = _conv(t, delta_w, delta_b, 'VALID')
        n, _, H, W = logits.shape
        logits = jnp.transpose(logits, (0, 2, 3, 1)).reshape(n, -1)
        deltas = jnp.transpose(deltas.reshape(n, _A, 4, H, W),
                               (0, 3, 4, 1, 2)).reshape(n, -1, 4)
        anc = jnp.asarray(anchors[lvl])
        k = min(_PRE_NMS_TOPK, int(anc.shape[0]))
        topv, topi = jax.lax.top_k(logits, k)
        dsel = jnp.take_along_axis(deltas, topi[:, :, None], axis=1)
        asel = anc[topi]
        boxes = jax.vmap(_decode)(asel, dsel)
        boxes = jnp.clip(boxes, 0.0, float(_IMG))
        kp = _round_up(k, _B)
        if kp != k:
            boxes = jnp.pad(boxes, ((0, 0), (0, kp - k), (0, 0)))
            topv = jnp.pad(topv, ((0, 0), (0, kp - k)),
                           constant_values=-jnp.inf)
        offs = jnp.full((kp,), float(lvl), jnp.float32) * (_IMG + 1.0)
        boxes_nms = boxes + offs[None, :, None]
        keep = _nms_keep(boxes_nms)
        masked = jnp.where(keep > 0.5, topv, -jnp.inf)
        all_boxes.append(boxes)
        all_scores.append(masked)
    boxes = jnp.concatenate(all_boxes, axis=1)
    scores = jnp.concatenate(all_scores, axis=1)
    out_s, topi = jax.lax.top_k(scores, _POST_NMS_TOPK)
    out_b = jnp.take_along_axis(boxes, topi[:, :, None], axis=1)
    return out_b, out_s
